# baseline (speedup 1.0000x reference)
import functools

import jax
import jax.numpy as jnp
from jax import lax
from jax.experimental import pallas as pl
from jax.experimental.pallas import tpu as pltpu
from jax.experimental.pallas import tpu_sc as plsc

N = 2
C = 128
D = H = W = 48
DHW = D * H * W
P = 100000

NC = 2
NS = 16
L = 16

PW = 6272
PB = PW * NS
NCHUNK = PW // L


def _axis_terms(c, d):
    g = 2.0 * c / (d - 1.0) - 1.0
    i = ((g + 1.0) * d - 1.0) / 2.0
    t = i.astype(jnp.int32)
    tf = t.astype(jnp.float32)
    i0 = jnp.where(tf > i, t - 1, t)
    i0f = i0.astype(jnp.float32)
    w1 = i - i0f
    w0 = (i0f + 1.0) - i
    i1 = i0 + 1
    v0 = (i0 >= 0) & (i0 <= d - 1)
    v1 = (i1 >= 0) & (i1 <= d - 1)
    w0 = jnp.where(v0, w0, 0.0)
    w1 = jnp.where(v1, w1, 0.0)
    i0c = jnp.clip(i0, 0, d - 1)
    i1c = jnp.clip(i1, 0, d - 1)
    return i0c, i1c, w0, w1


def _sc_body(table, coords, out, coords_v, idx_v, rows_v, out_v, wbuf, sem):
    batch = lax.axis_index("c")
    sub = lax.axis_index("s")

    pltpu.sync_copy(coords.at[batch, sub], coords_v)

    out_base = batch * PB + sub * PW

    def chunk_body(g, _):
        off = g * L
        x = coords_v[0, pl.ds(off, L)]
        y = coords_v[1, pl.ds(off, L)]
        z = coords_v[2, pl.ds(off, L)]
        x0, x1, wx0, wx1 = _axis_terms(x, W)
        y0, y1, wy0, wy1 = _axis_terms(y, H)
        z0, z1, wz0, wz1 = _axis_terms(z, D)

        base = batch * DHW
        corners = (
            (x0, y0, z0, wx0 * wy0 * wz0),
            (x1, y0, z0, wx1 * wy0 * wz0),
            (x0, y1, z0, wx0 * wy1 * wz0),
            (x1, y1, z0, wx1 * wy1 * wz0),
            (x0, y0, z1, wx0 * wy0 * wz1),
            (x1, y0, z1, wx1 * wy0 * wz1),
            (x0, y1, z1, wx0 * wy1 * wz1),
            (x1, y1, z1, wx1 * wy1 * wz1),
        )
        for ci, (xc, yc, zc, w) in enumerate(corners):
            idx_v[pl.ds(ci * L, L)] = base + (zc * H + yc) * W + xc
            wbuf[ci] = w

        pltpu.async_copy(table.at[idx_v], rows_v, sem).wait()

        w8 = [wbuf[ci] for ci in range(8)]

        def point_body(p, _):
            pvec = jnp.full((L,), p, dtype=jnp.int32)
            wb = [
                w8[ci].at[pvec].get(mode="promise_in_bounds")
                for ci in range(8)
            ]
            for j in range(C // L):
                acc = wb[0] * rows_v[p, pl.ds(j * L, L)]
                for ci in range(1, 8):
                    acc = acc + wb[ci] * rows_v[ci * L + p, pl.ds(j * L, L)]
                out_v[p, pl.ds(j * L, L)] = acc
            return 0

        lax.fori_loop(0, L, point_body, 0)

        pltpu.sync_copy(out_v, out.at[pl.ds(out_base + off, L)])
        return 0

    lax.fori_loop(0, NCHUNK, chunk_body, 0)


@jax.jit
def kernel(encoder_outputs, graph_coords):
    table = (
        encoder_outputs.reshape(N, C, DHW).transpose(0, 2, 1).reshape(N * DHW, C)
    )
    coords = graph_coords.reshape(N, P, 3)
    coords = jnp.pad(coords, ((0, 0), (0, PB - P), (0, 0)))
    coords = coords.reshape(N, NS, PW, 3).transpose(0, 1, 3, 2)

    mesh = plsc.VectorSubcoreMesh(core_axis_name="c", subcore_axis_name="s")
    run = pl.kernel(
        _sc_body,
        out_type=jax.ShapeDtypeStruct((N * PB, C), jnp.float32),
        mesh=mesh,
        scratch_types=[
            pltpu.VMEM((3, PW), jnp.float32),
            pltpu.VMEM((8 * L,), jnp.int32),
            pltpu.VMEM((8 * L, C), jnp.float32),
            pltpu.VMEM((L, C), jnp.float32),
            pltpu.VMEM((8, L), jnp.float32),
            pltpu.SemaphoreType.DMA,
        ],
    )
    out = run(table, coords)
    out = out.reshape(N, PB, C)[:, :P]
    return out.reshape(N, 1, 1, P, C)
